# Initial kernel scaffold; baseline (speedup 1.0000x reference)
import jax
import jax.numpy as jnp
from jax.experimental import pallas as pl


def kernel(input, index, value):
    raise NotImplementedError("write your pallas kernel here")


# speedup vs baseline: 84.3811x; 84.3811x over previous
import functools

import jax
import jax.numpy as jnp
from jax import lax
from jax.experimental import pallas as pl
from jax.experimental.pallas import tpu as pltpu
from jax.experimental.pallas import tpu_sc as plsc

R = 1024
I = 512
C = 256 * 128
L = 16
W = 64
G = W // L
NW = 32
CPW = C // NW
BLKS = CPW // W


def _scatter_body(in_hbm, idx_hbm, val_hbm, out_hbm, buf, idxbuf, valbuf):
    wid = lax.axis_index("s") * 2 + lax.axis_index("c")

    pltpu.sync_copy(val_hbm, valbuf)
    val = valbuf[...]
    cols = [lax.iota(jnp.int32, L) + g * L for g in range(G)]

    for b in range(BLKS):
        c0 = wid * CPW + b * W
        pltpu.sync_copy(in_hbm.at[:, pl.ds(c0, W)], buf)
        pltpu.sync_copy(idx_hbm.at[:, pl.ds(c0, W)], idxbuf)

        def body(i, carry):
            for g in range(G):
                rows = idxbuf[i, pl.ds(g * L, L)]
                plsc.store_scatter(buf, [rows, cols[g]], val)
            return carry

        lax.fori_loop(0, I, body, 0)
        pltpu.sync_copy(buf, out_hbm.at[:, pl.ds(c0, W)])


def kernel(input, index, value):
    inp = input.reshape(R, C)
    idx = index.astype(jnp.int32).reshape(I, C)
    val = jnp.full((L,), value, jnp.float32)

    mesh = plsc.VectorSubcoreMesh(core_axis_name="c", subcore_axis_name="s")
    run = functools.partial(
        pl.kernel,
        mesh=mesh,
        out_type=jax.ShapeDtypeStruct((R, C), jnp.float32),
        scratch_types=[
            pltpu.VMEM((R, W), jnp.float32),
            pltpu.VMEM((I, W), jnp.int32),
            pltpu.VMEM((L,), jnp.float32),
        ],
        compiler_params=pltpu.CompilerParams(
            use_tc_tiling_on_sc=False,
            needs_layout_passes=False,
        ),
    )(_scatter_body)
    out = run(inp, idx, val)
    return out.reshape(R, 256, 128)
